# Initial kernel scaffold; baseline (speedup 1.0000x reference)
import jax
import jax.numpy as jnp
from jax.experimental import pallas as pl


def kernel(z, codebook):
    raise NotImplementedError("write your pallas kernel here")


# speedup vs baseline: 1.0686x; 1.0686x over previous
import functools

import jax
import jax.numpy as jnp
from jax import lax
from jax.experimental import pallas as pl
from jax.experimental.pallas import tpu as pltpu
from jax.experimental.pallas import tpu_sc as plsc

_NUM_CODES = 8192
_DIM = 32
_COMMIT = 0.25

_BM = 1024
_BN = 4096

_NC = 2
_NS = 16
_NW = _NC * _NS


def _dist_argmin_body(z_ref, cbt_ref, zn_ref, en_ref, idx_ref, dsum_ref,
                      acc_bf, acc_idx, acc_val):
    i = pl.program_id(0)
    j = pl.program_id(1)
    nj = pl.num_programs(1)

    zv = z_ref[...]
    cv = cbt_ref[...]
    zn = zn_ref[...]
    en = en_ref[...]

    dot = lax.dot_general(zv, cv, (((1,), (0,)), ((), ())),
                          preferred_element_type=jnp.float32)
    dist = (zn + en) - 2.0 * dot

    loc_min = jnp.min(dist, axis=1, keepdims=True)
    iota = lax.broadcasted_iota(jnp.int32, (_BM, _BN), 1)
    loc_arg = jnp.min(jnp.where(dist == loc_min, iota, _BN),
                      axis=1, keepdims=True) + j * _BN

    @pl.when(j == 0)
    def _init():
        acc_bf[...] = jnp.full((_BM, 1), jnp.inf, jnp.bfloat16)
        acc_idx[...] = jnp.zeros((_BM, 1), jnp.int32)
        acc_val[...] = jnp.zeros((_BM, 1), jnp.float32)

    accv = acc_bf[...].astype(jnp.float32)
    keep = accv <= loc_min
    acc_idx[...] = jnp.where(keep, acc_idx[...], loc_arg)
    acc_val[...] = jnp.where(keep, acc_val[...], loc_min)
    acc_bf[...] = jnp.where(keep, accv, loc_min).astype(jnp.bfloat16)

    @pl.when(j == nj - 1)
    def _emit():
        idx_ref[...] = acc_idx[...].reshape(1, _BM, 1)
        part = jnp.sum(acc_val[...])

        @pl.when(i == 0)
        def _first():
            dsum_ref[0, 0] = part

        @pl.when(i > 0)
        def _acc():
            dsum_ref[0, 0] = dsum_ref[0, 0] + part


def _dist_argmin(z_flat, cbt, zn, en, interpret=False):
    m = z_flat.shape[0]
    grid = (m // _BM, _NUM_CODES // _BN)
    return pl.pallas_call(
        _dist_argmin_body,
        grid=grid,
        in_specs=[
            pl.BlockSpec((_BM, _DIM), lambda i, j: (i, 0)),
            pl.BlockSpec((_DIM, _BN), lambda i, j: (0, j)),
            pl.BlockSpec((_BM, 1), lambda i, j: (i, 0)),
            pl.BlockSpec((1, _BN), lambda i, j: (0, j)),
        ],
        out_specs=[
            pl.BlockSpec((1, _BM, 1), lambda i, j: (i, 0, 0)),
            pl.BlockSpec(memory_space=pltpu.SMEM),
        ],
        out_shape=[
            jax.ShapeDtypeStruct((m // _BM, _BM, 1), jnp.int32),
            jax.ShapeDtypeStruct((1, 1), jnp.float32),
        ],
        scratch_shapes=[
            pltpu.VMEM((_BM, 1), jnp.bfloat16),
            pltpu.VMEM((_BM, 1), jnp.int32),
            pltpu.VMEM((_BM, 1), jnp.float32),
        ],
        interpret=interpret,
    )(z_flat, cbt, zn, en)


def _sc_gather_st(codebook, idx2d, z_flat):
    m = z_flat.shape[0]
    bpw = m // _NW
    nch = bpw // 128

    mesh = plsc.VectorSubcoreMesh(core_axis_name="c", subcore_axis_name="s")

    @functools.partial(
        pl.kernel,
        mesh=mesh,
        out_type=jax.ShapeDtypeStruct((m, _DIM), jnp.float32),
        scratch_types=[
            pltpu.VMEM((nch, 128), jnp.int32),
            pltpu.VMEM((bpw, _DIM), jnp.float32),
            pltpu.VMEM((bpw, _DIM), jnp.float32),
            pltpu.SemaphoreType.DMA,
        ],
        compiler_params=pltpu.CompilerParams(use_tc_tiling_on_sc=False),
    )
    def body(cb_hbm, idx_hbm, z_hbm, out_hbm, idx_v, rows_v, z_v, sem):
        wid = lax.axis_index("s") * _NC + lax.axis_index("c")
        base = wid * bpw
        pltpu.sync_copy(idx_hbm.at[pl.ds(wid * nch, nch)], idx_v)
        pltpu.sync_copy(z_hbm.at[pl.ds(base, bpw)], z_v)
        for k in range(nch):
            pltpu.async_copy(cb_hbm.at[idx_v.at[k]],
                             rows_v.at[pl.ds(k * 128, 128)], sem).wait()

        def st_body(r, carry):
            for c in (0, 16):
                zv = z_v[r, pl.ds(c, 16)]
                qv = rows_v[r, pl.ds(c, 16)]
                rows_v[r, pl.ds(c, 16)] = zv + (qv - zv)
            return carry

        lax.fori_loop(0, bpw, st_body, 0)
        pltpu.sync_copy(rows_v, out_hbm.at[pl.ds(base, bpw)])

    return body(codebook, idx2d, z_flat)


def _zn_fused_order(z_flat):
    sq = z_flat * z_flat
    a = ((sq[:, 0:8] + sq[:, 8:16]) + sq[:, 16:24]) + sq[:, 24:32]
    b = a[:, 0:4] + a[:, 4:8]
    c = b[:, 0:2] + b[:, 2:4]
    return c[:, 0:1] + c[:, 1:2]


def kernel(z, codebook):
    b, n, dd = z.shape
    m = b * n
    z_flat = z.reshape(m, dd)
    cbt = codebook.T
    zn = _zn_fused_order(z_flat)
    en = jnp.sum(codebook ** 2, axis=1).reshape(1, _NUM_CODES)

    idx_blocks, dsum = _dist_argmin(z_flat, cbt, zn, en)
    indices_2d = idx_blocks.reshape(b, n)

    idx_chunks = idx_blocks.reshape(m // 128, 128)
    z_q_st = _sc_gather_st(codebook, idx_chunks, z_flat).reshape(b, n, dd)

    mse = dsum[0, 0] / jnp.float32(m * dd)
    codebook_loss = mse
    commit_loss = _COMMIT * mse
    vq_loss = codebook_loss + commit_loss
    return (z_q_st, indices_2d, vq_loss, codebook_loss, commit_loss)
